# baseline (speedup 1.0000x reference)
import functools

import jax
import jax.numpy as jnp
from jax import lax
from jax.experimental import pallas as pl
from jax.experimental.pallas import tpu as pltpu
from jax.experimental.pallas import tpu_sc as plsc

N_NODES = 10000
N_EDGES = 320000
D = 128

NC = 2
NS = 16
NW = NC * NS
L = 16

N_PAD = 10240
CHUNK = 128
EROWS = 2560
E_PAD = EROWS * CHUNK
ROWS_PER_W = EROWS // NW
NODES_PER_TILE = N_PAD // NS

_mesh = plsc.VectorSubcoreMesh(
    core_axis_name="c", subcore_axis_name="s", num_cores=NC, num_subcores=NS)


HROWS = 128
ROWS_PER_TILE_K1 = EROWS // NS


def _deg_body(dst_hbm, idx_hbm, z128_hbm, deg_hbm, dst_v, hist_v, idx_v, deg_sh):
    c = lax.axis_index("c")
    s = lax.axis_index("s")

    @pl.when(c == 0)
    def _():
        pltpu.sync_copy(
            dst_hbm.at[pl.ds(s * ROWS_PER_TILE_K1, ROWS_PER_TILE_K1)], dst_v)
        pltpu.sync_copy(z128_hbm, hist_v)
        pltpu.sync_copy(idx_hbm, idx_v)
        pltpu.sync_copy(z128_hbm.at[pl.ds(0, HROWS // NS)],
                        deg_sh.at[pl.ds(s * (HROWS // NS), HROWS // NS)])

        ones_vec = jnp.ones((L,), jnp.float32)

        def hist_body(i, carry):
            r = i // (CHUNK // L)
            k = i % (CHUNK // L)
            v = dst_v[r, pl.ds(k * L, L)]
            plsc.addupdate_scatter(hist_v, [v >> 7, v & 127], ones_vec)
            return carry
        lax.fori_loop(0, ROWS_PER_TILE_K1 * (CHUNK // L), hist_body, 0)
        plsc.subcore_barrier()

        pltpu.sync_copy(hist_v, deg_sh.at[idx_v.at[0]], add=True)
        plsc.subcore_barrier()

        pltpu.sync_copy(deg_sh.at[pl.ds(s * (HROWS // NS), HROWS // NS)],
                        deg_hbm.at[pl.ds(s * (HROWS // NS), HROWS // NS)])


def _make_deg_kernel(interpret=False):
    return pl.kernel(
        _deg_body,
        out_type=jax.ShapeDtypeStruct((HROWS, CHUNK), jnp.float32),
        mesh=_mesh,
        scratch_types=[
            pltpu.VMEM((ROWS_PER_TILE_K1, CHUNK), jnp.int32),
            pltpu.VMEM((HROWS, CHUNK), jnp.float32),
            pltpu.VMEM((1, CHUNK), jnp.int32),
            pltpu.VMEM_SHARED((HROWS, CHUNK), jnp.float32),
        ],
        compiler_params=pltpu.CompilerParams(needs_layout_passes=False),
        interpret=interpret,
    )


_deg_kernel = _make_deg_kernel()


ROWS_PER_TILE = EROWS // NS


def _edge_body_for(core, rows_per_tile, row_base):
    def body(g_hbm, src_hbm, dst_hbm, z128_hbm, acc_hbm,
             src_c, dst_c, rows2, acc_sh, gsem2, ssem2):
        c = lax.axis_index("c")
        s = lax.axis_index("s")

        @pl.when(c == core)
        def _():
            base = row_base + s * rows_per_tile
            pltpu.sync_copy(src_hbm.at[pl.ds(base, 8)], src_c.at[pl.ds(0, 8)])
            pltpu.sync_copy(dst_hbm.at[pl.ds(base, 8)], dst_c.at[pl.ds(0, 8)])
            pltpu.sync_copy(
                z128_hbm, acc_sh.at[pl.ds(s * NODES_PER_TILE, NODES_PER_TILE)])
            plsc.subcore_barrier()

            pltpu.async_copy(g_hbm.at[src_c.at[0]], rows2.at[0], gsem2.at[0])

            def edge_step(r, carry):
                p = r % 2
                q = (r + 1) % 2

                @pl.when(r >= 1)
                def _():
                    pltpu.make_async_copy(
                        g_hbm.at[pl.ds(0, CHUNK)], rows2.at[q],
                        ssem2.at[q]).wait()

                @pl.when(jnp.logical_and((r + 1) % 8 == 0,
                                         r + 1 < rows_per_tile))
                def _():
                    off = pl.multiple_of(base + r + 1, 8)
                    woff = pl.multiple_of((r + 1) % 16, 8)
                    pltpu.sync_copy(src_hbm.at[pl.ds(off, 8)],
                                    src_c.at[pl.ds(woff, 8)])
                    pltpu.sync_copy(dst_hbm.at[pl.ds(off, 8)],
                                    dst_c.at[pl.ds(woff, 8)])

                @pl.when(r + 1 < rows_per_tile)
                def _():
                    pltpu.async_copy(g_hbm.at[src_c.at[(r + 1) % 16]],
                                     rows2.at[q], gsem2.at[q])

                pltpu.make_async_copy(g_hbm.at[src_c.at[r % 16]], rows2.at[p],
                                      gsem2.at[p]).wait()
                pltpu.async_copy(rows2.at[p], acc_sh.at[dst_c.at[r % 16]],
                                 ssem2.at[p], add=True)
                return carry
            lax.fori_loop(0, rows_per_tile, edge_step, 0)
            pltpu.make_async_copy(
                g_hbm.at[pl.ds(0, CHUNK)], rows2.at[(rows_per_tile - 1) % 2],
                ssem2.at[(rows_per_tile - 1) % 2]).wait()
            plsc.subcore_barrier()

            pltpu.sync_copy(
                acc_sh.at[pl.ds(s * NODES_PER_TILE, NODES_PER_TILE)],
                acc_hbm.at[pl.ds(s * NODES_PER_TILE, NODES_PER_TILE)])
    return body


def _make_edge_kernel(core, rows_per_tile, row_base, interpret=False):
    return pl.kernel(
        _edge_body_for(core, rows_per_tile, row_base),
        out_type=jax.ShapeDtypeStruct((N_PAD, D), jnp.float32),
        mesh=_mesh,
        scratch_types=[
            pltpu.VMEM((16, CHUNK), jnp.int32),
            pltpu.VMEM((16, CHUNK), jnp.int32),
            pltpu.VMEM((2, CHUNK, D), jnp.float32),
            pltpu.VMEM_SHARED((N_PAD, D), jnp.float32),
            pltpu.SemaphoreType.DMA((2,)),
            pltpu.SemaphoreType.DMA((2,)),
        ],
        interpret=interpret,
    )


_edge_kernel_a = _make_edge_kernel(0, EROWS // 2 // NS, 0)
_edge_kernel_b = _make_edge_kernel(1, EROWS // 2 // NS, EROWS // 2)


def _matmul_body(x_ref, wt_ref, d_ref, g_ref):
    dis = lax.rsqrt(d_ref[...] + 1.0)
    h = jnp.dot(x_ref[...], wt_ref[...], preferred_element_type=jnp.float32)
    g_ref[...] = dis * h


_BLK = 256
_matmul = pl.pallas_call(
    _matmul_body,
    grid=(N_PAD // _BLK,),
    in_specs=[
        pl.BlockSpec((_BLK, D), lambda i: (i, 0)),
        pl.BlockSpec((D, D), lambda i: (0, 0)),
        pl.BlockSpec((_BLK, 1), lambda i: (i, 0)),
    ],
    out_specs=pl.BlockSpec((_BLK, D), lambda i: (i, 0)),
    out_shape=jax.ShapeDtypeStruct((N_PAD, D), jnp.float32),
)


def _combine_body(a0_ref, a1_ref, g_ref, d_ref, b_ref, out_ref):
    dis = lax.rsqrt(d_ref[...] + 1.0)
    out_ref[...] = dis * (a0_ref[...] + a1_ref[...] + g_ref[...]) + b_ref[...]


_combine = pl.pallas_call(
    _combine_body,
    grid=(N_PAD // _BLK,),
    in_specs=[
        pl.BlockSpec((_BLK, D), lambda i: (i, 0)),
        pl.BlockSpec((_BLK, D), lambda i: (i, 0)),
        pl.BlockSpec((_BLK, D), lambda i: (i, 0)),
        pl.BlockSpec((_BLK, 1), lambda i: (i, 0)),
        pl.BlockSpec((1, D), lambda i: (0, 0)),
    ],
    out_specs=pl.BlockSpec((_BLK, D), lambda i: (i, 0)),
    out_shape=jax.ShapeDtypeStruct((N_PAD, D), jnp.float32),
)


def kernel(x, edge_index, W, b):
    ei = edge_index.astype(jnp.int32)
    pad = jnp.full((E_PAD - N_EDGES,), N_PAD - 1, jnp.int32)
    src2d = jnp.concatenate([ei[0], pad]).reshape(EROWS, CHUNK)
    dst2d = jnp.concatenate([ei[1], pad]).reshape(EROWS, CHUNK)
    x_pad = jnp.pad(x, ((0, N_PAD - N_NODES), (0, 0)))

    idx128 = lax.iota(jnp.int32, CHUNK).reshape(1, CHUNK)
    z128 = jnp.zeros((NODES_PER_TILE, D), jnp.float32)

    deg = _deg_kernel(dst2d, idx128, z128[:HROWS])
    deg_col = deg.reshape(HROWS * CHUNK, 1)[:N_PAD]
    g = _matmul(x_pad, W.T, deg_col)
    acc_a = _edge_kernel_a(g, src2d, dst2d, z128)
    acc_b = _edge_kernel_b(g, src2d, dst2d, z128)
    out = _combine(acc_a, acc_b, g, deg_col, b.reshape(1, D))
    return out[:N_NODES]


# speedup vs baseline: 1.1471x; 1.1471x over previous
import functools

import jax
import jax.numpy as jnp
from jax import lax
from jax.experimental import pallas as pl
from jax.experimental.pallas import tpu as pltpu
from jax.experimental.pallas import tpu_sc as plsc

N_NODES = 10000
N_EDGES = 320000
D = 128

NC = 2
NS = 16
NW = NC * NS
L = 16

N_PAD = 10240
CHUNK = 128
EROWS = 2560
E_PAD = EROWS * CHUNK
ROWS_PER_W = EROWS // NW
NODES_PER_TILE = N_PAD // NS

_mesh = plsc.VectorSubcoreMesh(
    core_axis_name="c", subcore_axis_name="s", num_cores=NC, num_subcores=NS)


HROWS = 128
ROWS_PER_TILE_K1 = EROWS // NS


def _deg_body(dst_hbm, idx_hbm, z128_hbm, deg_hbm, dst_v, hist_v, idx_v, deg_sh):
    c = lax.axis_index("c")
    s = lax.axis_index("s")

    @pl.when(c == 0)
    def _():
        pltpu.sync_copy(
            dst_hbm.at[pl.ds(s * ROWS_PER_TILE_K1, ROWS_PER_TILE_K1)], dst_v)
        pltpu.sync_copy(z128_hbm, hist_v)
        pltpu.sync_copy(idx_hbm, idx_v)
        pltpu.sync_copy(z128_hbm.at[pl.ds(0, HROWS // NS)],
                        deg_sh.at[pl.ds(s * (HROWS // NS), HROWS // NS)])

        ones_vec = jnp.ones((L,), jnp.float32)

        def hist_body(i, carry):
            r = i // (CHUNK // L)
            k = i % (CHUNK // L)
            v = dst_v[r, pl.ds(k * L, L)]
            plsc.addupdate_scatter(hist_v, [v >> 7, v & 127], ones_vec)
            return carry
        lax.fori_loop(0, ROWS_PER_TILE_K1 * (CHUNK // L), hist_body, 0)
        plsc.subcore_barrier()

        pltpu.sync_copy(hist_v, deg_sh.at[idx_v.at[0]], add=True)
        plsc.subcore_barrier()

        pltpu.sync_copy(deg_sh.at[pl.ds(s * (HROWS // NS), HROWS // NS)],
                        deg_hbm.at[pl.ds(s * (HROWS // NS), HROWS // NS)])


def _make_deg_kernel(interpret=False):
    return pl.kernel(
        _deg_body,
        out_type=jax.ShapeDtypeStruct((HROWS, CHUNK), jnp.float32),
        mesh=_mesh,
        scratch_types=[
            pltpu.VMEM((ROWS_PER_TILE_K1, CHUNK), jnp.int32),
            pltpu.VMEM((HROWS, CHUNK), jnp.float32),
            pltpu.VMEM((1, CHUNK), jnp.int32),
            pltpu.VMEM_SHARED((HROWS, CHUNK), jnp.float32),
        ],
        compiler_params=pltpu.CompilerParams(needs_layout_passes=False),
        interpret=interpret,
    )


_deg_kernel = _make_deg_kernel()


ROWS_PER_TILE = EROWS // NS


def _edge_body_for(core, rows_per_tile, row_base):
    def body(g_hbm, src_hbm, dst_hbm, z128_hbm, acc_hbm,
             src_c, dst_c, rows2, acc_sh, gsem2, ssem2):
        c = lax.axis_index("c")
        s = lax.axis_index("s")

        @pl.when(c == core)
        def _():
            base = row_base + s * rows_per_tile
            pltpu.sync_copy(src_hbm.at[pl.ds(base, 8)], src_c.at[pl.ds(0, 8)])
            pltpu.sync_copy(dst_hbm.at[pl.ds(base, 8)], dst_c.at[pl.ds(0, 8)])
            pltpu.sync_copy(
                z128_hbm, acc_sh.at[pl.ds(s * NODES_PER_TILE, NODES_PER_TILE)])
            plsc.subcore_barrier()

            pltpu.async_copy(g_hbm.at[src_c.at[0]], rows2.at[0], gsem2.at[0])

            def edge_step(r, carry):
                p = r % 2
                q = (r + 1) % 2

                @pl.when(r >= 1)
                def _():
                    pltpu.make_async_copy(
                        g_hbm.at[pl.ds(0, CHUNK)], rows2.at[q],
                        ssem2.at[q]).wait()

                @pl.when(jnp.logical_and((r + 1) % 8 == 0,
                                         r + 1 < rows_per_tile))
                def _():
                    off = pl.multiple_of(base + r + 1, 8)
                    woff = pl.multiple_of((r + 1) % 16, 8)
                    pltpu.sync_copy(src_hbm.at[pl.ds(off, 8)],
                                    src_c.at[pl.ds(woff, 8)])
                    pltpu.sync_copy(dst_hbm.at[pl.ds(off, 8)],
                                    dst_c.at[pl.ds(woff, 8)])

                @pl.when(r + 1 < rows_per_tile)
                def _():
                    pltpu.async_copy(g_hbm.at[src_c.at[(r + 1) % 16]],
                                     rows2.at[q], gsem2.at[q])

                pltpu.make_async_copy(g_hbm.at[src_c.at[r % 16]], rows2.at[p],
                                      gsem2.at[p]).wait()
                pltpu.async_copy(rows2.at[p], acc_sh.at[dst_c.at[r % 16]],
                                 ssem2.at[p], add=True)
                return carry
            lax.fori_loop(0, rows_per_tile, edge_step, 0)
            pltpu.make_async_copy(
                g_hbm.at[pl.ds(0, CHUNK)], rows2.at[(rows_per_tile - 1) % 2],
                ssem2.at[(rows_per_tile - 1) % 2]).wait()
            plsc.subcore_barrier()

            pltpu.sync_copy(
                acc_sh.at[pl.ds(s * NODES_PER_TILE, NODES_PER_TILE)],
                acc_hbm.at[pl.ds(s * NODES_PER_TILE, NODES_PER_TILE)])
    return body


def _make_edge_kernel(core, rows_per_tile, row_base, interpret=False):
    return pl.kernel(
        _edge_body_for(core, rows_per_tile, row_base),
        out_type=jax.ShapeDtypeStruct((N_PAD, D), jnp.float32),
        mesh=_mesh,
        scratch_types=[
            pltpu.VMEM((16, CHUNK), jnp.int32),
            pltpu.VMEM((16, CHUNK), jnp.int32),
            pltpu.VMEM((2, CHUNK, D), jnp.float32),
            pltpu.VMEM_SHARED((N_PAD, D), jnp.float32),
            pltpu.SemaphoreType.DMA((2,)),
            pltpu.SemaphoreType.DMA((2,)),
        ],
        interpret=interpret,
    )


_edge_kernel_a = _make_edge_kernel(0, EROWS // NS, 0)


def _matmul_body(x_ref, wt_ref, d_ref, g_ref):
    dis = lax.rsqrt(d_ref[...] + 1.0)
    h = jnp.dot(x_ref[...], wt_ref[...], preferred_element_type=jnp.float32)
    g_ref[...] = dis * h


_BLK = 256
_matmul = pl.pallas_call(
    _matmul_body,
    grid=(N_PAD // _BLK,),
    in_specs=[
        pl.BlockSpec((_BLK, D), lambda i: (i, 0)),
        pl.BlockSpec((D, D), lambda i: (0, 0)),
        pl.BlockSpec((_BLK, 1), lambda i: (i, 0)),
    ],
    out_specs=pl.BlockSpec((_BLK, D), lambda i: (i, 0)),
    out_shape=jax.ShapeDtypeStruct((N_PAD, D), jnp.float32),
)


def _combine_body(a0_ref, g_ref, d_ref, b_ref, out_ref):
    dis = lax.rsqrt(d_ref[...] + 1.0)
    out_ref[...] = dis * (a0_ref[...] + g_ref[...]) + b_ref[...]


_combine = pl.pallas_call(
    _combine_body,
    grid=(N_PAD // _BLK,),
    in_specs=[
        pl.BlockSpec((_BLK, D), lambda i: (i, 0)),
        pl.BlockSpec((_BLK, D), lambda i: (i, 0)),
        pl.BlockSpec((_BLK, 1), lambda i: (i, 0)),
        pl.BlockSpec((1, D), lambda i: (0, 0)),
    ],
    out_specs=pl.BlockSpec((_BLK, D), lambda i: (i, 0)),
    out_shape=jax.ShapeDtypeStruct((N_PAD, D), jnp.float32),
)


def kernel(x, edge_index, W, b):
    ei = edge_index.astype(jnp.int32)
    pad = jnp.full((E_PAD - N_EDGES,), N_PAD - 1, jnp.int32)
    src2d = jnp.concatenate([ei[0], pad]).reshape(EROWS, CHUNK)
    dst2d = jnp.concatenate([ei[1], pad]).reshape(EROWS, CHUNK)
    x_pad = jnp.pad(x, ((0, N_PAD - N_NODES), (0, 0)))

    idx128 = lax.iota(jnp.int32, CHUNK).reshape(1, CHUNK)
    z128 = jnp.zeros((NODES_PER_TILE, D), jnp.float32)

    deg = _deg_kernel(dst2d, idx128, z128[:HROWS])
    deg_col = deg.reshape(HROWS * CHUNK, 1)[:N_PAD]
    g = _matmul(x_pad, W.T, deg_col)
    acc = _edge_kernel_a(g, src2d, dst2d, z128)
    out = _combine(acc, g, deg_col, b.reshape(1, D))
    return out[:N_NODES]


# speedup vs baseline: 1.1607x; 1.0118x over previous
import functools

import jax
import jax.numpy as jnp
from jax import lax
from jax.experimental import pallas as pl
from jax.experimental.pallas import tpu as pltpu
from jax.experimental.pallas import tpu_sc as plsc

N_NODES = 10000
N_EDGES = 320000
D = 128

NC = 2
NS = 16
NW = NC * NS
L = 16

N_PAD = 10240
CHUNK = 128
EROWS = 2560
E_PAD = EROWS * CHUNK
ROWS_PER_W = EROWS // NW
NODES_PER_TILE = N_PAD // NS

_mesh = plsc.VectorSubcoreMesh(
    core_axis_name="c", subcore_axis_name="s", num_cores=NC, num_subcores=NS)


HROWS = 128
ROWS_PER_TILE_K1 = EROWS // NS


def _deg_body(dst_hbm, idx_hbm, z128_hbm, deg_hbm, dst_v, hist_v, idx_v, deg_sh):
    c = lax.axis_index("c")
    s = lax.axis_index("s")

    @pl.when(c == 0)
    def _():
        pltpu.sync_copy(
            dst_hbm.at[pl.ds(s * ROWS_PER_TILE_K1, ROWS_PER_TILE_K1)], dst_v)
        pltpu.sync_copy(z128_hbm, hist_v)
        pltpu.sync_copy(idx_hbm, idx_v)
        pltpu.sync_copy(z128_hbm.at[pl.ds(0, HROWS // NS)],
                        deg_sh.at[pl.ds(s * (HROWS // NS), HROWS // NS)])

        ones_vec = jnp.ones((L,), jnp.float32)

        def hist_body(i, carry):
            r = i // (CHUNK // L)
            k = i % (CHUNK // L)
            v = dst_v[r, pl.ds(k * L, L)]
            plsc.addupdate_scatter(hist_v, [v >> 7, v & 127], ones_vec)
            return carry
        lax.fori_loop(0, ROWS_PER_TILE_K1 * (CHUNK // L), hist_body, 0)
        plsc.subcore_barrier()

        pltpu.sync_copy(hist_v, deg_sh.at[idx_v.at[0]], add=True)
        plsc.subcore_barrier()

        pltpu.sync_copy(deg_sh.at[pl.ds(s * (HROWS // NS), HROWS // NS)],
                        deg_hbm.at[pl.ds(s * (HROWS // NS), HROWS // NS)])


def _make_deg_kernel(interpret=False):
    return pl.kernel(
        _deg_body,
        out_type=jax.ShapeDtypeStruct((HROWS, CHUNK), jnp.float32),
        mesh=_mesh,
        scratch_types=[
            pltpu.VMEM((ROWS_PER_TILE_K1, CHUNK), jnp.int32),
            pltpu.VMEM((HROWS, CHUNK), jnp.float32),
            pltpu.VMEM((1, CHUNK), jnp.int32),
            pltpu.VMEM_SHARED((HROWS, CHUNK), jnp.float32),
        ],
        compiler_params=pltpu.CompilerParams(needs_layout_passes=False),
        interpret=interpret,
    )


_deg_kernel = _make_deg_kernel()


ROWS_PER_TILE = EROWS // NS


def _edge_body_for(core, rows_per_tile, row_base):
    def body(g_hbm, src_hbm, dst_hbm, z128_hbm, acc_hbm,
             src_c, dst_c, rows2, acc_sh, gsem2, ssem2):
        c = lax.axis_index("c")
        s = lax.axis_index("s")

        @pl.when(c == core)
        def _():
            base = row_base + s * rows_per_tile
            pltpu.sync_copy(src_hbm.at[pl.ds(base, 16)],
                            src_c.at[pl.ds(0, 16)])
            pltpu.sync_copy(dst_hbm.at[pl.ds(base, 16)],
                            dst_c.at[pl.ds(0, 16)])
            pltpu.sync_copy(
                z128_hbm, acc_sh.at[pl.ds(s * NODES_PER_TILE, NODES_PER_TILE)])
            plsc.subcore_barrier()

            pltpu.async_copy(g_hbm.at[src_c.at[0]], rows2.at[0], gsem2.at[0])

            def edge_step(r, carry):
                p = r % 2
                q = (r + 1) % 2

                @pl.when(r >= 1)
                def _():
                    pltpu.make_async_copy(
                        g_hbm.at[pl.ds(0, CHUNK)], rows2.at[q],
                        ssem2.at[q]).wait()

                @pl.when(jnp.logical_and((r + 1) % 16 == 0,
                                         r + 1 < rows_per_tile))
                def _():
                    off = pl.multiple_of(base + r + 1, 8)
                    woff = pl.multiple_of((r + 1) % 32, 8)
                    pltpu.sync_copy(src_hbm.at[pl.ds(off, 16)],
                                    src_c.at[pl.ds(woff, 16)])
                    pltpu.sync_copy(dst_hbm.at[pl.ds(off, 16)],
                                    dst_c.at[pl.ds(woff, 16)])

                @pl.when(r + 1 < rows_per_tile)
                def _():
                    pltpu.async_copy(g_hbm.at[src_c.at[(r + 1) % 32]],
                                     rows2.at[q], gsem2.at[q])

                pltpu.make_async_copy(g_hbm.at[src_c.at[r % 32]], rows2.at[p],
                                      gsem2.at[p]).wait()
                pltpu.async_copy(rows2.at[p], acc_sh.at[dst_c.at[r % 32]],
                                 ssem2.at[p], add=True)
                return carry
            lax.fori_loop(0, rows_per_tile, edge_step, 0)
            pltpu.make_async_copy(
                g_hbm.at[pl.ds(0, CHUNK)], rows2.at[(rows_per_tile - 1) % 2],
                ssem2.at[(rows_per_tile - 1) % 2]).wait()
            plsc.subcore_barrier()

            pltpu.sync_copy(
                acc_sh.at[pl.ds(s * NODES_PER_TILE, NODES_PER_TILE)],
                acc_hbm.at[pl.ds(s * NODES_PER_TILE, NODES_PER_TILE)])
    return body


def _make_edge_kernel(core, rows_per_tile, row_base, interpret=False):
    return pl.kernel(
        _edge_body_for(core, rows_per_tile, row_base),
        out_type=jax.ShapeDtypeStruct((N_PAD, D), jnp.float32),
        mesh=_mesh,
        scratch_types=[
            pltpu.VMEM((32, CHUNK), jnp.int32),
            pltpu.VMEM((32, CHUNK), jnp.int32),
            pltpu.VMEM((2, CHUNK, D), jnp.float32),
            pltpu.VMEM_SHARED((N_PAD, D), jnp.float32),
            pltpu.SemaphoreType.DMA((2,)),
            pltpu.SemaphoreType.DMA((2,)),
        ],
        interpret=interpret,
    )


_edge_kernel_a = _make_edge_kernel(0, EROWS // NS, 0)


def _matmul_body(x_ref, wt_ref, d_ref, g_ref):
    dis = lax.rsqrt(d_ref[...] + 1.0)
    h = jnp.dot(x_ref[...], wt_ref[...], preferred_element_type=jnp.float32)
    g_ref[...] = dis * h


_BLK = 400
_matmul = pl.pallas_call(
    _matmul_body,
    grid=(N_NODES // _BLK,),
    in_specs=[
        pl.BlockSpec((_BLK, D), lambda i: (i, 0)),
        pl.BlockSpec((D, D), lambda i: (0, 0)),
        pl.BlockSpec((_BLK, 1), lambda i: (i, 0)),
    ],
    out_specs=pl.BlockSpec((_BLK, D), lambda i: (i, 0)),
    out_shape=jax.ShapeDtypeStruct((N_NODES, D), jnp.float32),
)


def _combine_body(a0_ref, g_ref, d_ref, b_ref, out_ref):
    dis = lax.rsqrt(d_ref[...] + 1.0)
    out_ref[...] = dis * (a0_ref[...] + g_ref[...]) + b_ref[...]


_combine = pl.pallas_call(
    _combine_body,
    grid=(N_NODES // _BLK,),
    in_specs=[
        pl.BlockSpec((_BLK, D), lambda i: (i, 0)),
        pl.BlockSpec((_BLK, D), lambda i: (i, 0)),
        pl.BlockSpec((_BLK, 1), lambda i: (i, 0)),
        pl.BlockSpec((1, D), lambda i: (0, 0)),
    ],
    out_specs=pl.BlockSpec((_BLK, D), lambda i: (i, 0)),
    out_shape=jax.ShapeDtypeStruct((N_NODES, D), jnp.float32),
)


def kernel(x, edge_index, W, b):
    ei = edge_index.astype(jnp.int32)
    src2d = jnp.concatenate(
        [ei[0], jnp.zeros((E_PAD - N_EDGES,), jnp.int32)]).reshape(EROWS, CHUNK)
    dst2d = jnp.concatenate(
        [ei[1], jnp.full((E_PAD - N_EDGES,), N_PAD - 1, jnp.int32)]
    ).reshape(EROWS, CHUNK)

    idx128 = lax.iota(jnp.int32, CHUNK).reshape(1, CHUNK)
    z128 = jnp.zeros((NODES_PER_TILE, D), jnp.float32)

    deg = _deg_kernel(dst2d, idx128, z128[:HROWS])
    deg_col = deg.reshape(HROWS * CHUNK, 1)[:N_NODES]
    g = _matmul(x, W.T, deg_col)
    acc = _edge_kernel_a(g, src2d, dst2d, z128)
    return _combine(acc, g, deg_col, b.reshape(1, D))
